# baseline (speedup 1.0000x reference)
import functools

import jax
import jax.numpy as jnp
from jax.experimental import pallas as pl


def _smap_kernel(x_ref, y_ref, z_ref, r_ref, ray_ref, o_ref,
                 *, TH, H, W):
    h = pl.program_id(1)
    h0 = h * TH
    top_i = (h0 - 1 + H) % H
    bot_i = (h0 + TH) % H

    def padded(ref):
        top = ref[0, pl.ds(top_i, 1), :]
        mid = ref[0, pl.ds(h0, TH), :]
        bot = ref[0, pl.ds(bot_i, 1), :]
        return jnp.concatenate([top, mid, bot], axis=0)

    xp = padded(x_ref)
    yp = padded(y_ref)
    zp = padded(z_ref)

    xc = xp[1:TH + 1]
    yc = yp[1:TH + 1]
    zc = zp[1:TH + 1]
    rr = r_ref[0, pl.ds(h0, TH), :]

    bx = ray_ref[0, 0, pl.ds(h0, TH), :] * zc
    by = ray_ref[0, 1, pl.ds(h0, TH), :] * zc
    bz = ray_ref[0, 2, pl.ds(h0, TH), :] * zc

    best_d = None
    best_i = None
    s = 0
    for dh in (-1, 0, 1):
        rs = 1 - dh
        xs = xp[rs:rs + TH]
        ys = yp[rs:rs + TH]
        zs = zp[rs:rs + TH]
        for dw in (-1, 0, 1):
            nx = jnp.roll(xs, dw, axis=1) if dw else xs
            ny = jnp.roll(ys, dw, axis=1) if dw else ys
            nz = jnp.roll(zs, dw, axis=1) if dw else zs
            dx = nx - bx
            dy = ny - by
            dz = nz - bz
            d = (dx * dx + dy * dy) + dz * dz
            if s == 0:
                best_d = d
                best_i = jnp.zeros(d.shape, jnp.int32)
            else:
                better = d < best_d
                best_d = jnp.where(better, d, best_d)
                best_i = jnp.where(better, s, best_i)
            s += 1

    rgt = rr > 0.5
    valid = rgt & (zc > 0.0)
    idx_eff = jnp.where(valid, best_i, 4)
    zero = jnp.zeros_like(xc)
    for s in range(9):
        m = idx_eff == s
        mx = m & rgt
        o_ref[0, 4 * s + 0] = jnp.where(mx, xc, zero)
        o_ref[0, 4 * s + 1] = jnp.where(mx, yc, zero)
        o_ref[0, 4 * s + 2] = jnp.where(mx, zc, zero)
        o_ref[0, 4 * s + 3] = jnp.where(m, rr, zero)


def kernel(x_value, y_value, z_value, r_mask, panels, original_size,
           camera_matrix_inv):
    B, C, _, H, W = x_value.shape
    BC = B * C
    x = x_value.reshape(BC, H, W)
    y = y_value.reshape(BC, H, W)
    z = z_value.reshape(BC, H, W)
    r = r_mask.reshape(BC, H, W)

    u = panels[:, :, 0:1]
    v = panels[:, :, 1:2]
    pix = jnp.concatenate([u, v, jnp.ones_like(u)], axis=2)
    ray = jnp.einsum('ij,bcjhw->bcihw', camera_matrix_inv, pix)
    rayf = ray.reshape(BC, 3, H, W)

    TH = 64
    NH = H // TH

    plane_spec = pl.BlockSpec((1, H, W), lambda bc, h: (bc, 0, 0))
    ray_spec = pl.BlockSpec((1, 3, H, W), lambda bc, h: (bc, 0, 0, 0))
    out = pl.pallas_call(
        functools.partial(_smap_kernel, TH=TH, H=H, W=W),
        grid=(BC, NH),
        in_specs=[plane_spec] * 4 + [ray_spec],
        out_specs=pl.BlockSpec((1, 36, TH, W), lambda bc, h: (bc, 0, h, 0)),
        out_shape=jax.ShapeDtypeStruct((BC, 36, H, W), jnp.float32),
    )(x, y, z, r, rayf)
    return out.reshape(B, C, 3, 3, 4, H, W)


# speedup vs baseline: 1.0448x; 1.0448x over previous
import functools

import jax
import jax.numpy as jnp
from jax.experimental import pallas as pl


def _smap_kernel(x_ref, y_ref, z_ref, r_ref, ray_ref, o_ref,
                 *, TH, H, W):
    h = pl.program_id(1)
    h0 = h * TH
    top_i = (h0 - 1 + H) % H
    bot_i = (h0 + TH) % H

    def padded(ref):
        top = ref[0, pl.ds(top_i, 1), :]
        mid = ref[0, pl.ds(h0, TH), :]
        bot = ref[0, pl.ds(bot_i, 1), :]
        return jnp.concatenate([top, mid, bot], axis=0)

    xp = padded(x_ref)
    yp = padded(y_ref)
    zp = padded(z_ref)

    xc = xp[1:TH + 1]
    yc = yp[1:TH + 1]
    zc = zp[1:TH + 1]
    rr = r_ref[0, pl.ds(h0, TH), :]

    bx = ray_ref[0, 0, pl.ds(h0, TH), :] * zc
    by = ray_ref[0, 1, pl.ds(h0, TH), :] * zc
    bz = ray_ref[0, 2, pl.ds(h0, TH), :] * zc

    best_d = None
    best_i = None
    s = 0
    for dh in (-1, 0, 1):
        rs = 1 - dh
        xs = xp[rs:rs + TH]
        ys = yp[rs:rs + TH]
        zs = zp[rs:rs + TH]
        for dw in (-1, 0, 1):
            nx = jnp.roll(xs, dw, axis=1) if dw else xs
            ny = jnp.roll(ys, dw, axis=1) if dw else ys
            nz = jnp.roll(zs, dw, axis=1) if dw else zs
            dx = nx - bx
            dy = ny - by
            dz = nz - bz
            d = (dx * dx + dy * dy) + dz * dz
            if s == 0:
                best_d = d
                best_i = jnp.zeros(d.shape, jnp.int32)
            else:
                better = d < best_d
                best_d = jnp.where(better, d, best_d)
                best_i = jnp.where(better, s, best_i)
            s += 1

    rgt = rr > 0.5
    valid = rgt & (zc > 0.0)
    idx_eff = jnp.where(valid, best_i, 4)
    zero = jnp.zeros_like(xc)
    for s in range(9):
        m = idx_eff == s
        mx = m & rgt
        o_ref[0, 4 * s + 0] = jnp.where(mx, xc, zero)
        o_ref[0, 4 * s + 1] = jnp.where(mx, yc, zero)
        o_ref[0, 4 * s + 2] = jnp.where(mx, zc, zero)
        o_ref[0, 4 * s + 3] = jnp.where(m, rr, zero)


def kernel(x_value, y_value, z_value, r_mask, panels, original_size,
           camera_matrix_inv):
    B, C, _, H, W = x_value.shape
    BC = B * C
    x = x_value.reshape(BC, H, W)
    y = y_value.reshape(BC, H, W)
    z = z_value.reshape(BC, H, W)
    r = r_mask.reshape(BC, H, W)

    u = panels[:, :, 0:1]
    v = panels[:, :, 1:2]
    pix = jnp.concatenate([u, v, jnp.ones_like(u)], axis=2)
    ray = jnp.einsum('ij,bcjhw->bcihw', camera_matrix_inv, pix)
    rayf = ray.reshape(BC, 3, H, W)

    TH = 128
    NH = H // TH

    plane_spec = pl.BlockSpec((1, H, W), lambda bc, h: (bc, 0, 0))
    ray_spec = pl.BlockSpec((1, 3, H, W), lambda bc, h: (bc, 0, 0, 0))
    out = pl.pallas_call(
        functools.partial(_smap_kernel, TH=TH, H=H, W=W),
        grid=(BC, NH),
        in_specs=[plane_spec] * 4 + [ray_spec],
        out_specs=pl.BlockSpec((1, 36, TH, W), lambda bc, h: (bc, 0, h, 0)),
        out_shape=jax.ShapeDtypeStruct((BC, 36, H, W), jnp.float32),
    )(x, y, z, r, rayf)
    return out.reshape(B, C, 3, 3, 4, H, W)


# speedup vs baseline: 1.7036x; 1.6306x over previous
import functools

import jax
import jax.numpy as jnp
from jax.experimental import pallas as pl
from jax.experimental.pallas import tpu as pltpu


def _smap_kernel(cam_ref, x_ref, y_ref, z_ref, r_ref, pan_ref, o_ref,
                 *, TH, H, W):
    h = pl.program_id(1)
    h0 = h * TH
    top_i = (h0 - 1 + H) % H
    bot_i = (h0 + TH) % H

    def padded(ref):
        top = ref[0, pl.ds(top_i, 1), :]
        mid = ref[0, pl.ds(h0, TH), :]
        bot = ref[0, pl.ds(bot_i, 1), :]
        return jnp.concatenate([top, mid, bot], axis=0)

    xp = padded(x_ref)
    yp = padded(y_ref)
    zp = padded(z_ref)

    xc = xp[1:TH + 1]
    yc = yp[1:TH + 1]
    zc = zp[1:TH + 1]
    rr = r_ref[0, pl.ds(h0, TH), :]
    ub = pan_ref[0, 0, pl.ds(h0, TH), :].astype(jnp.float32)
    vb = pan_ref[0, 1, pl.ds(h0, TH), :].astype(jnp.float32)

    k00 = cam_ref[0, 0]
    k01 = cam_ref[0, 1]
    k02 = cam_ref[0, 2]
    k10 = cam_ref[1, 0]
    k11 = cam_ref[1, 1]
    k12 = cam_ref[1, 2]
    k20 = cam_ref[2, 0]
    k21 = cam_ref[2, 1]
    k22 = cam_ref[2, 2]

    bx = ((k00 * ub + k01 * vb) + k02) * zc
    by = ((k10 * ub + k11 * vb) + k12) * zc
    bz = ((k20 * ub + k21 * vb) + k22) * zc

    best_d = None
    best_i = None
    s = 0
    for dh in (-1, 0, 1):
        rs = 1 - dh
        xs = xp[rs:rs + TH]
        ys = yp[rs:rs + TH]
        zs = zp[rs:rs + TH]
        for dw in (-1, 0, 1):
            nx = jnp.roll(xs, dw, axis=1) if dw else xs
            ny = jnp.roll(ys, dw, axis=1) if dw else ys
            nz = jnp.roll(zs, dw, axis=1) if dw else zs
            dx = nx - bx
            dy = ny - by
            dz = nz - bz
            d = (dx * dx + dy * dy) + dz * dz
            if s == 0:
                best_d = d
                best_i = jnp.zeros(d.shape, jnp.int32)
            else:
                better = d < best_d
                best_d = jnp.where(better, d, best_d)
                best_i = jnp.where(better, s, best_i)
            s += 1

    rgt = rr > 0.5
    valid = rgt & (zc > 0.0)
    idx_eff = jnp.where(valid, best_i, 4)
    zero = jnp.zeros_like(xc)
    for s in range(9):
        m = idx_eff == s
        mx = m & rgt
        o_ref[0, 4 * s + 0] = jnp.where(mx, xc, zero)
        o_ref[0, 4 * s + 1] = jnp.where(mx, yc, zero)
        o_ref[0, 4 * s + 2] = jnp.where(mx, zc, zero)
        o_ref[0, 4 * s + 3] = jnp.where(m, rr, zero)


def kernel(x_value, y_value, z_value, r_mask, panels, original_size,
           camera_matrix_inv):
    B, C, _, H, W = x_value.shape
    BC = B * C
    x = x_value.reshape(BC, H, W)
    y = y_value.reshape(BC, H, W)
    z = z_value.reshape(BC, H, W)
    r = r_mask.reshape(BC, H, W)
    pan = panels.reshape(BC, 2, H, W).astype(jnp.bfloat16)
    bits = jax.lax.bitcast_convert_type(camera_matrix_inv, jnp.uint32)
    bits = (bits + jnp.uint32(0x7FFF)
            + ((bits >> jnp.uint32(16)) & jnp.uint32(1))) & jnp.uint32(0xFFFF0000)
    camb = jax.lax.bitcast_convert_type(bits, jnp.float32)

    TH = 128
    NH = H // TH

    plane_spec = pl.BlockSpec((1, H, W), lambda bc, h: (bc, 0, 0))
    pan_spec = pl.BlockSpec((1, 2, H, W), lambda bc, h: (bc, 0, 0, 0))
    out = pl.pallas_call(
        functools.partial(_smap_kernel, TH=TH, H=H, W=W),
        grid=(BC, NH),
        in_specs=[pl.BlockSpec(memory_space=pltpu.SMEM)] + [plane_spec] * 4
        + [pan_spec],
        out_specs=pl.BlockSpec((1, 36, TH, W), lambda bc, h: (bc, 0, h, 0)),
        out_shape=jax.ShapeDtypeStruct((BC, 36, H, W), jnp.float32),
    )(camb, x, y, z, r, pan)
    return out.reshape(B, C, 3, 3, 4, H, W)


# speedup vs baseline: 1.7039x; 1.0002x over previous
import functools

import jax
import jax.numpy as jnp
from jax.experimental import pallas as pl
from jax.experimental.pallas import tpu as pltpu


def _smap_kernel(cam_ref, x_ref, y_ref, z_ref, r_ref, pan_ref, o_ref,
                 *, TH, H, W):
    h = pl.program_id(1)
    h0 = h * TH
    top_i = (h0 - 1 + H) % H
    bot_i = (h0 + TH) % H

    def padded(ref):
        top = ref[0, pl.ds(top_i, 1), :]
        mid = ref[0, pl.ds(h0, TH), :]
        bot = ref[0, pl.ds(bot_i, 1), :]
        return jnp.concatenate([top, mid, bot], axis=0)

    xp = padded(x_ref)
    yp = padded(y_ref)
    zp = padded(z_ref)

    xc = xp[1:TH + 1]
    yc = yp[1:TH + 1]
    zc = zp[1:TH + 1]
    rr = r_ref[0, pl.ds(h0, TH), :]
    ub = pan_ref[0, 0, pl.ds(h0, TH), :].astype(jnp.float32)
    vb = pan_ref[0, 1, pl.ds(h0, TH), :].astype(jnp.float32)

    k00 = cam_ref[0, 0]
    k01 = cam_ref[0, 1]
    k02 = cam_ref[0, 2]
    k10 = cam_ref[1, 0]
    k11 = cam_ref[1, 1]
    k12 = cam_ref[1, 2]
    k20 = cam_ref[2, 0]
    k21 = cam_ref[2, 1]
    k22 = cam_ref[2, 2]

    bx = ((k00 * ub + k01 * vb) + k02) * zc
    by = ((k10 * ub + k11 * vb) + k12) * zc
    bz = ((k20 * ub + k21 * vb) + k22) * zc

    best_d = None
    best_i = None
    s = 0
    for dh in (-1, 0, 1):
        rs = 1 - dh
        xs = xp[rs:rs + TH]
        ys = yp[rs:rs + TH]
        zs = zp[rs:rs + TH]
        for dw in (-1, 0, 1):
            nx = jnp.roll(xs, dw, axis=1) if dw else xs
            ny = jnp.roll(ys, dw, axis=1) if dw else ys
            nz = jnp.roll(zs, dw, axis=1) if dw else zs
            dx = nx - bx
            dy = ny - by
            dz = nz - bz
            d = (dx * dx + dy * dy) + dz * dz
            if s == 0:
                best_d = d
                best_i = jnp.zeros(d.shape, jnp.int32)
            else:
                better = d < best_d
                best_d = jnp.where(better, d, best_d)
                best_i = jnp.where(better, s, best_i)
            s += 1

    rgt = rr > 0.5
    valid = rgt & (zc > 0.0)
    idx_eff = jnp.where(valid, best_i, 4)
    zero = jnp.zeros_like(xc)
    for s in range(9):
        m = idx_eff == s
        mx = m & rgt
        o_ref[0, 4 * s + 0] = jnp.where(mx, xc, zero)
        o_ref[0, 4 * s + 1] = jnp.where(mx, yc, zero)
        o_ref[0, 4 * s + 2] = jnp.where(mx, zc, zero)
        o_ref[0, 4 * s + 3] = jnp.where(m, rr, zero)


def kernel(x_value, y_value, z_value, r_mask, panels, original_size,
           camera_matrix_inv):
    B, C, _, H, W = x_value.shape
    BC = B * C
    x = x_value.reshape(BC, H, W)
    y = y_value.reshape(BC, H, W)
    z = z_value.reshape(BC, H, W)
    r = r_mask.reshape(BC, H, W)
    pan = panels.reshape(BC, 2, H, W).astype(jnp.bfloat16)
    bits = jax.lax.bitcast_convert_type(camera_matrix_inv, jnp.uint32)
    bits = (bits + jnp.uint32(0x7FFF)
            + ((bits >> jnp.uint32(16)) & jnp.uint32(1))) & jnp.uint32(0xFFFF0000)
    camb = jax.lax.bitcast_convert_type(bits, jnp.float32)

    TH = 128
    NH = H // TH

    plane_spec = pl.BlockSpec((1, H, W), lambda bc, h: (bc, 0, 0))
    pan_spec = pl.BlockSpec((1, 2, H, W), lambda bc, h: (bc, 0, 0, 0))
    out = pl.pallas_call(
        functools.partial(_smap_kernel, TH=TH, H=H, W=W),
        grid=(BC, NH),
        in_specs=[pl.BlockSpec(memory_space=pltpu.SMEM)] + [plane_spec] * 4
        + [pan_spec],
        out_specs=pl.BlockSpec((1, 36, TH, W), lambda bc, h: (bc, 0, h, 0)),
        out_shape=jax.ShapeDtypeStruct((BC, 36, H, W), jnp.float32),
        compiler_params=pltpu.CompilerParams(
            dimension_semantics=("parallel", "arbitrary")),
    )(camb, x, y, z, r, pan)
    return out.reshape(B, C, 3, 3, 4, H, W)


# speedup vs baseline: 1.7110x; 1.0041x over previous
import functools

import jax
import jax.numpy as jnp
from jax.experimental import pallas as pl
from jax.experimental.pallas import tpu as pltpu


def _smap_kernel(cam_ref, x_ref, y_ref, z_ref, r_ref, pan_ref, o_ref,
                 *, TH, H, W):
    h = pl.program_id(1)
    h0 = h * TH
    top_i = (h0 - 1 + H) % H
    bot_i = (h0 + TH) % H

    def padded(ref):
        top = ref[0, pl.ds(top_i, 1), :]
        mid = ref[0, pl.ds(h0, TH), :]
        bot = ref[0, pl.ds(bot_i, 1), :]
        return jnp.concatenate([top, mid, bot], axis=0)

    xp = padded(x_ref)
    yp = padded(y_ref)
    zp = padded(z_ref)

    xc = xp[1:TH + 1]
    yc = yp[1:TH + 1]
    zc = zp[1:TH + 1]
    rr = r_ref[0, pl.ds(h0, TH), :]
    ub = pan_ref[0, 0, pl.ds(h0, TH), :].astype(jnp.float32)
    vb = pan_ref[0, 1, pl.ds(h0, TH), :].astype(jnp.float32)

    k00 = cam_ref[0, 0]
    k01 = cam_ref[0, 1]
    k02 = cam_ref[0, 2]
    k10 = cam_ref[1, 0]
    k11 = cam_ref[1, 1]
    k12 = cam_ref[1, 2]
    k20 = cam_ref[2, 0]
    k21 = cam_ref[2, 1]
    k22 = cam_ref[2, 2]

    bx = ((k00 * ub + k01 * vb) + k02) * zc
    by = ((k10 * ub + k11 * vb) + k12) * zc
    bz = ((k20 * ub + k21 * vb) + k22) * zc

    best_d = None
    best_i = None
    s = 0
    for dh in (-1, 0, 1):
        rs = 1 - dh
        xs = xp[rs:rs + TH]
        ys = yp[rs:rs + TH]
        zs = zp[rs:rs + TH]
        for dw in (-1, 0, 1):
            nx = jnp.roll(xs, dw, axis=1) if dw else xs
            ny = jnp.roll(ys, dw, axis=1) if dw else ys
            nz = jnp.roll(zs, dw, axis=1) if dw else zs
            dx = nx - bx
            dy = ny - by
            dz = nz - bz
            d = (dx * dx + dy * dy) + dz * dz
            if s == 0:
                best_d = d
                best_i = jnp.zeros(d.shape, jnp.int32)
            else:
                better = d < best_d
                best_d = jnp.where(better, d, best_d)
                best_i = jnp.where(better, s, best_i)
            s += 1

    rgt = rr > 0.5
    valid = rgt & (zc > 0.0)
    idx_eff = jnp.where(valid, best_i, 4)
    zero = jnp.zeros_like(xc)
    xm = jnp.where(rgt, xc, zero)
    ym = jnp.where(rgt, yc, zero)
    zm = jnp.where(rgt, zc, zero)
    for s in range(9):
        m = idx_eff == s
        o_ref[0, 4 * s + 0] = jnp.where(m, xm, zero)
        o_ref[0, 4 * s + 1] = jnp.where(m, ym, zero)
        o_ref[0, 4 * s + 2] = jnp.where(m, zm, zero)
        o_ref[0, 4 * s + 3] = jnp.where(m, rr, zero)


def kernel(x_value, y_value, z_value, r_mask, panels, original_size,
           camera_matrix_inv):
    B, C, _, H, W = x_value.shape
    BC = B * C
    x = x_value.reshape(BC, H, W)
    y = y_value.reshape(BC, H, W)
    z = z_value.reshape(BC, H, W)
    r = r_mask.reshape(BC, H, W)
    pan = panels.reshape(BC, 2, H, W).astype(jnp.bfloat16)
    bits = jax.lax.bitcast_convert_type(camera_matrix_inv, jnp.uint32)
    bits = (bits + jnp.uint32(0x7FFF)
            + ((bits >> jnp.uint32(16)) & jnp.uint32(1))) & jnp.uint32(0xFFFF0000)
    camb = jax.lax.bitcast_convert_type(bits, jnp.float32)

    TH = 128
    NH = H // TH

    plane_spec = pl.BlockSpec((1, H, W), lambda bc, h: (bc, 0, 0))
    pan_spec = pl.BlockSpec((1, 2, H, W), lambda bc, h: (bc, 0, 0, 0))
    out = pl.pallas_call(
        functools.partial(_smap_kernel, TH=TH, H=H, W=W),
        grid=(BC, NH),
        in_specs=[pl.BlockSpec(memory_space=pltpu.SMEM)] + [plane_spec] * 4
        + [pan_spec],
        out_specs=pl.BlockSpec((1, 36, TH, W), lambda bc, h: (bc, 0, h, 0)),
        out_shape=jax.ShapeDtypeStruct((BC, 36, H, W), jnp.float32),
        compiler_params=pltpu.CompilerParams(
            dimension_semantics=("parallel", "arbitrary")),
    )(camb, x, y, z, r, pan)
    return out.reshape(B, C, 3, 3, 4, H, W)
